# baseline (speedup 1.0000x reference)
import functools

import jax
import jax.numpy as jnp
from jax import lax
from jax.experimental import pallas as pl
from jax.experimental.pallas import tpu as pltpu
from jax.experimental.pallas import tpu_sc as plsc

NUM_EMB = 1_000_000
DIM = 64
NC = 2
NS = 16
NW = NC * NS
BATCH = 4096
SEQ = 200
NCHUNK = 4
CB = BATCH // NCHUNK
BPW = CB // NW
NBUF = 2
NGRP = BPW // NBUF


def _make_gather():
    mesh = plsc.VectorSubcoreMesh(
        core_axis_name="c", subcore_axis_name="s", num_cores=NC, num_subcores=NS
    )

    @functools.partial(
        pl.kernel,
        out_type=jax.ShapeDtypeStruct((CB, SEQ, DIM), jnp.float32),
        mesh=mesh,
        scratch_types=[
            pltpu.VMEM((BPW, SEQ), jnp.int32),
            [pltpu.VMEM((SEQ, DIM), jnp.float32) for _ in range(NBUF)],
            pltpu.SemaphoreType.DMA((NBUF,)),
            pltpu.SemaphoreType.DMA((NBUF,)),
        ],
    )
    def body(table_hbm, idx_hbm, out_hbm, idx_v, bufs, gsem, osem):
        wid = lax.axis_index("s") * NC + lax.axis_index("c")
        base = wid * BPW
        pltpu.sync_copy(idx_hbm.at[pl.ds(base, BPW)], idx_v)

        def fire(local, b):
            for j16 in range(12):
                v = idx_v[local, pl.ds(j16 * 16, 16)]
                for j in range(16):
                    pltpu.async_copy(
                        table_hbm.at[v[j]], bufs[b].at[j16 * 16 + j], gsem.at[b]
                    )
            v = idx_v[local, pl.ds(SEQ - 16, 16)]
            for j in range(8, 16):
                pltpu.async_copy(
                    table_hbm.at[v[j]], bufs[b].at[SEQ - 16 + j], gsem.at[b]
                )

        def wait_gather(b):
            pltpu.make_async_copy(
                table_hbm.at[pl.ds(0, SEQ)], bufs[b], gsem.at[b]
            ).wait()

        for b in range(NBUF):
            fire(b, b)

        @pl.loop(0, NGRP - 1)
        def _(grp):
            for b in range(NBUF):
                local = grp * NBUF + b
                wait_gather(b)
                pltpu.async_copy(bufs[b], out_hbm.at[base + local], osem.at[b])
            for b in range(NBUF):
                local = grp * NBUF + b
                pltpu.make_async_copy(
                    bufs[b], out_hbm.at[base + local], osem.at[b]
                ).wait()
                fire(local + NBUF, b)

        last = (NGRP - 1) * NBUF
        for b in range(NBUF):
            wait_gather(b)
            pltpu.async_copy(bufs[b], out_hbm.at[base + last + b], osem.at[b])
        for b in range(NBUF):
            pltpu.make_async_copy(
                bufs[b], out_hbm.at[base + last + b], osem.at[b]
            ).wait()

    return body


def kernel(embeddings, token_ids):
    table_rm = jax.lax.optimization_barrier(embeddings.T).T
    ids = token_ids.astype(jnp.int32)
    gather = _make_gather()
    outs = [
        gather(table_rm, lax.slice_in_dim(ids, i * CB, (i + 1) * CB, axis=0))
        for i in range(NCHUNK)
    ]
    return jnp.concatenate(outs, axis=0)


# speedup vs baseline: 1.1133x; 1.1133x over previous
import functools

import jax
import jax.numpy as jnp
from jax import lax
from jax.experimental import pallas as pl
from jax.experimental.pallas import tpu as pltpu
from jax.experimental.pallas import tpu_sc as plsc

NUM_EMB = 1_000_000
DIM = 64
NC = 2
NS = 16
NW = NC * NS
BATCH = 4096
SEQ = 200
NCHUNK = 4
CB = BATCH // NCHUNK
BPW = CB // NW
NBUF = 2
NGRP = BPW // NBUF


def _make_gather():
    mesh = plsc.VectorSubcoreMesh(
        core_axis_name="c", subcore_axis_name="s", num_cores=NC, num_subcores=NS
    )

    @functools.partial(
        pl.kernel,
        out_type=jax.ShapeDtypeStruct((CB, SEQ, DIM), jnp.float32),
        mesh=mesh,
        scratch_types=[
            pltpu.VMEM((BPW, SEQ), jnp.int32),
            [pltpu.VMEM((SEQ, DIM), jnp.float32) for _ in range(NBUF)],
            pltpu.SemaphoreType.DMA((NBUF,)),
            pltpu.SemaphoreType.DMA((NBUF,)),
        ],
    )
    def body(table_hbm, idx_hbm, out_hbm, idx_v, bufs, gsem, osem):
        wid = lax.axis_index("s") * NC + lax.axis_index("c")
        base = wid * BPW
        pltpu.sync_copy(idx_hbm.at[pl.ds(base, BPW)], idx_v)

        def fire(local, b):
            for j16 in range(12):
                v = idx_v[local, pl.ds(j16 * 16, 16)]
                for j in range(16):
                    pltpu.async_copy(
                        table_hbm.at[v[j]], bufs[b].at[j16 * 16 + j], gsem.at[b]
                    )
            v = idx_v[local, pl.ds(SEQ - 16, 16)]
            for j in range(8, 16):
                pltpu.async_copy(
                    table_hbm.at[v[j]], bufs[b].at[SEQ - 16 + j], gsem.at[b]
                )

        def wait_gather(b):
            pltpu.make_async_copy(
                table_hbm.at[pl.ds(0, SEQ)], bufs[b], gsem.at[b]
            ).wait()

        for b in range(NBUF):
            fire(b, b)

        @pl.loop(0, NGRP - 1)
        def _(grp):
            for b in range(NBUF):
                local = grp * NBUF + b
                wait_gather(b)
                pltpu.async_copy(bufs[b], out_hbm.at[base + local], osem.at[b])
            for b in range(NBUF):
                local = grp * NBUF + b
                pltpu.make_async_copy(
                    bufs[b], out_hbm.at[base + local], osem.at[b]
                ).wait()
                fire(local + NBUF, b)

        last = (NGRP - 1) * NBUF
        for b in range(NBUF):
            wait_gather(b)
            pltpu.async_copy(bufs[b], out_hbm.at[base + last + b], osem.at[b])
        for b in range(NBUF):
            pltpu.make_async_copy(
                bufs[b], out_hbm.at[base + last + b], osem.at[b]
            ).wait()

    return body


def kernel(embeddings, token_ids):
    table_rm = jax.lax.optimization_barrier(embeddings.T).T
    ids = token_ids.astype(jnp.int32)
    gather = _make_gather()
    from jax.experimental.layout import Layout, with_layout_constraint

    fmt = Layout(major_to_minor=(1, 2, 0))
    acc = with_layout_constraint(
        jnp.zeros((BATCH, SEQ, DIM), jnp.float32), fmt
    )
    for i in range(NCHUNK):
        o = gather(table_rm, lax.slice_in_dim(ids, i * CB, (i + 1) * CB, axis=0))
        acc = lax.dynamic_update_slice(acc, o, (i * CB, 0, 0))
        acc = with_layout_constraint(acc, fmt)
    return acc


# speedup vs baseline: 1.2654x; 1.1366x over previous
import functools

import jax
import jax.numpy as jnp
from jax import lax
from jax.experimental import pallas as pl
from jax.experimental.pallas import tpu as pltpu
from jax.experimental.pallas import tpu_sc as plsc

NUM_EMB = 1_000_000
DIM = 64
NC = 2
NS = 16
NW = NC * NS
BATCH = 4096
SEQ = 200
BPW = BATCH // NW
NBUF = 2
NGRP = BPW // NBUF


def _emb_gather(table, idx):
    mesh = plsc.VectorSubcoreMesh(
        core_axis_name="c", subcore_axis_name="s", num_cores=NC, num_subcores=NS
    )

    @functools.partial(
        pl.kernel,
        out_type=jax.ShapeDtypeStruct((BATCH, SEQ, DIM), jnp.float32),
        mesh=mesh,
        scratch_types=[
            pltpu.VMEM((BPW, SEQ), jnp.int32),
            [pltpu.VMEM((SEQ, DIM), jnp.float32) for _ in range(NBUF)],
            pltpu.SemaphoreType.DMA((NBUF,)),
            pltpu.SemaphoreType.DMA((NBUF,)),
        ],
    )
    def body(table_hbm, idx_hbm, out_hbm, idx_v, bufs, gsem, osem):
        wid = lax.axis_index("s") * NC + lax.axis_index("c")
        base = wid * BPW
        pltpu.sync_copy(idx_hbm.at[pl.ds(base, BPW)], idx_v)

        def fire(local, b):
            for j16 in range(12):
                v = idx_v[local, pl.ds(j16 * 16, 16)]
                for j in range(16):
                    pltpu.async_copy(
                        table_hbm.at[v[j]], bufs[b].at[j16 * 16 + j], gsem.at[b]
                    )
            v = idx_v[local, pl.ds(SEQ - 16, 16)]
            for j in range(8, 16):
                pltpu.async_copy(
                    table_hbm.at[v[j]], bufs[b].at[SEQ - 16 + j], gsem.at[b]
                )

        def wait_gather(b):
            for j in range(SEQ):
                pltpu.make_async_copy(
                    table_hbm.at[j], bufs[b].at[j], gsem.at[b]
                ).wait()

        for b in range(NBUF):
            fire(b, b)

        @pl.loop(0, NGRP - 1)
        def _(grp):
            for b in range(NBUF):
                local = grp * NBUF + b
                wait_gather(b)
                pltpu.async_copy(bufs[b], out_hbm.at[base + local], osem.at[b])
            for b in range(NBUF):
                local = grp * NBUF + b
                pltpu.make_async_copy(
                    bufs[b], out_hbm.at[base + local], osem.at[b]
                ).wait()
                fire(local + NBUF, b)

        last = (NGRP - 1) * NBUF
        for b in range(NBUF):
            wait_gather(b)
            pltpu.async_copy(bufs[b], out_hbm.at[base + last + b], osem.at[b])
        for b in range(NBUF):
            pltpu.make_async_copy(
                bufs[b], out_hbm.at[base + last + b], osem.at[b]
            ).wait()

    return body(table, idx)


def kernel(embeddings, token_ids):
    table_rm = jax.lax.optimization_barrier(embeddings.T).T
    return _emb_gather(table_rm, token_ids.astype(jnp.int32))


# speedup vs baseline: 1.4373x; 1.1358x over previous
import functools

import jax
import jax.numpy as jnp
from jax import lax
from jax.experimental import pallas as pl
from jax.experimental.pallas import tpu as pltpu
from jax.experimental.pallas import tpu_sc as plsc

NUM_EMB = 1_000_000
DIM = 64
NC = 2
NS = 16
NW = NC * NS
BATCH = 4096
SEQ = 200
BPW = BATCH // NW
NBUF = 2
NGRP = BPW // NBUF


def _emb_gather(table, idx):
    mesh = plsc.VectorSubcoreMesh(
        core_axis_name="c", subcore_axis_name="s", num_cores=NC, num_subcores=NS
    )

    @functools.partial(
        pl.kernel,
        out_type=jax.ShapeDtypeStruct((BATCH, SEQ, DIM), jnp.float32),
        mesh=mesh,
        scratch_types=[
            pltpu.VMEM((BPW, SEQ), jnp.int32),
            [pltpu.VMEM((SEQ, DIM), jnp.float32) for _ in range(NBUF)],
            pltpu.SemaphoreType.DMA((NBUF,)),
            pltpu.SemaphoreType.DMA((NBUF,)),
        ],
    )
    def body(table_hbm, idx_hbm, out_hbm, idx_v, bufs, gsem, osem):
        wid = lax.axis_index("s") * NC + lax.axis_index("c")
        base = wid * BPW
        pltpu.sync_copy(idx_hbm.at[pl.ds(base, BPW)], idx_v)

        def fire(local, b):
            for j16 in range(12):
                v = idx_v[local, pl.ds(j16 * 16, 16)]
                for j in range(16):
                    pltpu.async_copy(
                        table_hbm.at[v[j]], bufs[b].at[j16 * 16 + j], gsem.at[b]
                    )
            v = idx_v[local, pl.ds(SEQ - 16, 16)]
            for j in range(8, 16):
                pltpu.async_copy(
                    table_hbm.at[v[j]], bufs[b].at[SEQ - 16 + j], gsem.at[b]
                )

        def wait_gather(b):
            for j in range(SEQ):
                pltpu.make_async_copy(
                    table_hbm.at[j], bufs[b].at[j], gsem.at[b]
                ).wait()

        for b in range(NBUF):
            fire(b, b)

        @pl.loop(0, NGRP - 1)
        def _(grp):
            for b in range(NBUF):
                local = grp * NBUF + b
                wait_gather(b)
                pltpu.async_copy(bufs[b], out_hbm.at[base + local], osem.at[b])
            for b in range(NBUF):
                local = grp * NBUF + b
                pltpu.make_async_copy(
                    bufs[b], out_hbm.at[base + local], osem.at[b]
                ).wait()
                fire(local + NBUF, b)

        last = (NGRP - 1) * NBUF
        for b in range(NBUF):
            wait_gather(b)
            pltpu.async_copy(bufs[b], out_hbm.at[base + last + b], osem.at[b])
        for b in range(NBUF):
            pltpu.make_async_copy(
                bufs[b], out_hbm.at[base + last + b], osem.at[b]
            ).wait()

    return body(table, idx)


def kernel(embeddings, token_ids):
    table_rm = jax.lax.optimization_barrier(embeddings.T).T
    out = _emb_gather(table_rm, token_ids.astype(jnp.int32))
    return jax.lax.optimization_barrier(out.transpose(1, 2, 0)).transpose(2, 0, 1)
